# Initial kernel scaffold; baseline (speedup 1.0000x reference)
import functools

import jax
import jax.numpy as jnp
from jax import lax
from jax.experimental import pallas as pl
from jax.experimental.pallas import tpu as pltpu
from jax.experimental.pallas import tpu_sc as plsc

N = 10000
E = 320000
D = 128
H = 128
C = 16
G = 128
K = 3

NC = 2
NS = 16
NW = NC * NS

CH = 128
CPW = -(-E // (NW * CH))
E_PAD = NW * CH * CPW
N_PAD = 10016
SUB_ROWS = N_PAD // NS
DUMMY_ROW = N
DW = 16

_mesh = plsc.VectorSubcoreMesh(core_axis_name="c", subcore_axis_name="s")


def _sc_degree(rows_pad, zeros_g, ones_ch):
    @functools.partial(
        pl.kernel,
        out_type=jax.ShapeDtypeStruct((NC, N_PAD, DW), jnp.float32),
        mesh=_mesh,
        scratch_types=[
            pltpu.VMEM_SHARED((N_PAD, DW), jnp.float32),
            pltpu.VMEM((CH,), jnp.int32),
            pltpu.VMEM((CH, DW), jnp.float32),
        ],
    )
    def k(rows_hbm, zeros_hbm, ones_hbm, out_hbm, acc, rowv, onesv):
        c = lax.axis_index("c")
        s = lax.axis_index("s")
        w = s * NC + c
        pltpu.sync_copy(ones_hbm, onesv)
        pltpu.sync_copy(zeros_hbm, acc.at[pl.ds(s * SUB_ROWS, SUB_ROWS)])
        plsc.subcore_barrier()

        base0 = w * (CPW * CH)

        @pl.loop(0, CPW)
        def _(i):
            pltpu.sync_copy(rows_hbm.at[pl.ds(base0 + i * CH, CH)], rowv)
            pltpu.sync_copy(onesv, acc.at[rowv], add=True)

        plsc.subcore_barrier()
        pltpu.sync_copy(acc.at[pl.ds(s * SUB_ROWS, SUB_ROWS)],
                        out_hbm.at[c, pl.ds(s * SUB_ROWS, SUB_ROWS)])

    return k(rows_pad, zeros_g, ones_ch)


def _sc_propagate(h, rows_pad, cols_pad, zeros_d):
    @functools.partial(
        pl.kernel,
        out_type=jax.ShapeDtypeStruct((NC, N_PAD, D), jnp.float32),
        mesh=_mesh,
        scratch_types=[
            pltpu.VMEM_SHARED((N_PAD, D), jnp.float32),
            pltpu.VMEM((CH,), jnp.int32),
            pltpu.VMEM((CH,), jnp.int32),
            pltpu.VMEM((CH, D), jnp.float32),
            pltpu.SemaphoreType.DMA,
        ],
    )
    def k(h_hbm, rows_hbm, cols_hbm, zeros_hbm, out_hbm, acc, colv, rowv, rowsb, sem):
        c = lax.axis_index("c")
        s = lax.axis_index("s")
        w = s * NC + c
        pltpu.sync_copy(zeros_hbm, acc.at[pl.ds(s * SUB_ROWS, SUB_ROWS)])
        plsc.subcore_barrier()

        base0 = w * (CPW * CH)

        @pl.loop(0, CPW)
        def _(i):
            base = base0 + i * CH
            pltpu.sync_copy(cols_hbm.at[pl.ds(base, CH)], colv)
            pltpu.sync_copy(rows_hbm.at[pl.ds(base, CH)], rowv)
            pltpu.async_copy(h_hbm.at[colv], rowsb, sem).wait()
            pltpu.sync_copy(rowsb, acc.at[rowv], add=True)

        plsc.subcore_barrier()
        pltpu.sync_copy(acc.at[pl.ds(s * SUB_ROWS, SUB_ROWS)],
                        out_hbm.at[c, pl.ds(s * SUB_ROWS, SUB_ROWS)])

    return k(h, rows_pad, cols_pad, zeros_d)


BN = 1000


def _tc_scale(t_parts, h, deg_parts):
    def body(t0, t1, h_ref, d0, d1, o_ref):
        d = d0[0, :, :1] + d1[0, :, :1] + 1.0
        o_ref[...] = (t0[0] + t1[0] + h_ref[...]) / d

    return pl.pallas_call(
        body,
        grid=(N // BN,),
        in_specs=[
            pl.BlockSpec((1, BN, D), lambda i: (0, i, 0)),
            pl.BlockSpec((1, BN, D), lambda i: (1, i, 0)),
            pl.BlockSpec((BN, D), lambda i: (i, 0)),
            pl.BlockSpec((1, BN, DW), lambda i: (0, i, 0)),
            pl.BlockSpec((1, BN, DW), lambda i: (1, i, 0)),
        ],
        out_specs=pl.BlockSpec((BN, D), lambda i: (i, 0)),
        out_shape=jax.ShapeDtypeStruct((N, D), jnp.float32),
    )(t_parts, t_parts, h, deg_parts, deg_parts)


def _tc_mlp_pool(t_parts, h, deg_parts, batch3, W1, b1, W2, b2, Wc, bc):
    nblk = N // BN

    def body(t0, t1, h_ref, d0, d1, b_ref, W1r, b1r, W2r, b2r, Wcr, bcr,
             o_ref, accr, cntr):
        i = pl.program_id(0)

        @pl.when(i == 0)
        def _():
            accr[...] = jnp.zeros_like(accr)
            cntr[...] = jnp.zeros_like(cntr)

        d = d0[0, :, :1] + d1[0, :, :1] + 1.0
        h3 = (t0[0] + t1[0] + h_ref[...]) / d
        a = jnp.dot(h3, W1r[...], preferred_element_type=jnp.float32) + b1r[...]
        a = jnp.maximum(a, 0.0)
        bvals = b_ref[...].reshape(1, BN)
        onehot_t = (lax.broadcasted_iota(jnp.int32, (G, BN), 0) == bvals
                    ).astype(jnp.float32)
        accr[...] += jnp.dot(onehot_t, a, preferred_element_type=jnp.float32)
        cntr[...] += jnp.sum(onehot_t, axis=1, keepdims=True)

        @pl.when(i == nblk - 1)
        def _():
            pooled = accr[...] / jnp.maximum(cntr[...], 1.0)
            p2 = jnp.dot(pooled, W2r[...], preferred_element_type=jnp.float32) + b2r[...]
            o_ref[...] = jnp.dot(p2, Wcr[...], preferred_element_type=jnp.float32) + bcr[...]

    return pl.pallas_call(
        body,
        grid=(nblk,),
        in_specs=[
            pl.BlockSpec((1, BN, D), lambda i: (0, i, 0)),
            pl.BlockSpec((1, BN, D), lambda i: (1, i, 0)),
            pl.BlockSpec((BN, D), lambda i: (i, 0)),
            pl.BlockSpec((1, BN, DW), lambda i: (0, i, 0)),
            pl.BlockSpec((1, BN, DW), lambda i: (1, i, 0)),
            pl.BlockSpec((1, 1, BN), lambda i: (i, 0, 0)),
            pl.BlockSpec((D, H), lambda i: (0, 0)),
            pl.BlockSpec((1, H), lambda i: (0, 0)),
            pl.BlockSpec((H, H), lambda i: (0, 0)),
            pl.BlockSpec((1, H), lambda i: (0, 0)),
            pl.BlockSpec((H, C), lambda i: (0, 0)),
            pl.BlockSpec((1, C), lambda i: (0, 0)),
        ],
        out_specs=pl.BlockSpec((G, C), lambda i: (0, 0)),
        out_shape=jax.ShapeDtypeStruct((G, C), jnp.float32),
        scratch_shapes=[
            pltpu.VMEM((G, H), jnp.float32),
            pltpu.VMEM((G, 1), jnp.float32),
        ],
    )(t_parts, t_parts, h, deg_parts, deg_parts, batch3,
      W1, b1.reshape(1, H), W2, b2.reshape(1, H), Wc, bc.reshape(1, C))


def kernel(x, edge_index, batch, W1, b1, W2, b2, Wc, bc):
    rows = edge_index[0]
    cols = edge_index[1]
    pad = E_PAD - E
    rows_pad = jnp.concatenate([rows, jnp.full((pad,), DUMMY_ROW, jnp.int32)])
    cols_pad = jnp.concatenate([cols, jnp.zeros((pad,), jnp.int32)])
    zeros_d = jnp.zeros((SUB_ROWS, D), jnp.float32)
    zeros_g = jnp.zeros((SUB_ROWS, DW), jnp.float32)
    ones_ch = jnp.ones((CH, DW), jnp.float32)
    batch3 = batch.reshape(N // BN, 1, BN)

    deg_parts = _sc_degree(rows_pad, zeros_g, ones_ch)
    h = x
    t_parts = None
    for step in range(K):
        t_parts = _sc_propagate(h, rows_pad, cols_pad, zeros_d)
        if step < K - 1:
            h = _tc_scale(t_parts, h, deg_parts)
    return _tc_mlp_pool(t_parts, h, deg_parts, batch3, W1, b1, W2, b2, Wc, bc)


# speedup vs baseline: 7.3205x; 7.3205x over previous
import functools

import jax
import jax.numpy as jnp
from jax import lax
from jax.experimental import pallas as pl
from jax.experimental.pallas import tpu as pltpu
from jax.experimental.pallas import tpu_sc as plsc

N = 10000
E = 320000
D = 128
H = 128
C = 16
G = 128
K = 3

NC = 2
NS = 16
NW = NC * NS

CH = 128
CPW = -(-E // (NW * CH))
E_PAD = NW * CH * CPW
N_PAD = 10240
SUB_ROWS = N_PAD // NS
DUMMY_ROW = N

@functools.cache
def _mesh():
    return plsc.VectorSubcoreMesh(core_axis_name="c", subcore_axis_name="s",
                                  num_cores=NC, num_subcores=NS)


@functools.cache
def _cp():
    import dataclasses
    cp = pltpu.CompilerParams()
    if "needs_layout_passes" in pltpu.CompilerParams.__dataclass_fields__:
        cp = dataclasses.replace(cp, needs_layout_passes=False)
    return cp


def _sc_degree(rows_pad):
    @functools.partial(
        pl.kernel,
        out_type=jax.ShapeDtypeStruct((NC, N_PAD), jnp.float32),
        mesh=_mesh(),
        compiler_params=_cp(),
        scratch_types=[
            pltpu.VMEM_SHARED((NS, N_PAD), jnp.float32),
            pltpu.VMEM((CH,), jnp.int32),
            pltpu.VMEM((N_PAD,), jnp.float32),
            pltpu.VMEM((NS, SUB_ROWS), jnp.float32),
        ],
    )
    def k(rows_hbm, out_hbm, stage, rowv, hist, gath):
        c = lax.axis_index("c")
        s = lax.axis_index("s")
        w = s * NC + c

        @pl.loop(0, N_PAD, step=16)
        def _(i):
            hist[pl.ds(i, 16)] = jnp.zeros((16,), jnp.float32)

        ones = jnp.ones((16,), jnp.float32)
        base0 = w * (CPW * CH)

        @pl.loop(0, CPW)
        def _(i):
            pltpu.sync_copy(rows_hbm.at[pl.ds(base0 + i * CH, CH)], rowv)

            @pl.loop(0, CH, step=16)
            def _(j):
                plsc.addupdate_scatter(hist, [rowv[pl.ds(j, 16)]], ones)

        pltpu.sync_copy(hist, stage.at[s])
        plsc.subcore_barrier()
        pltpu.sync_copy(stage.at[:, pl.ds(s * SUB_ROWS, SUB_ROWS)], gath)

        @pl.loop(0, SUB_ROWS, step=16)
        def _(i):
            acc16 = gath[0, pl.ds(i, 16)]
            for t in range(1, NS):
                acc16 = acc16 + gath[t, pl.ds(i, 16)]
            hist[pl.ds(i, 16)] = acc16

        pltpu.sync_copy(hist.at[pl.ds(0, SUB_ROWS)],
                        out_hbm.at[c, pl.ds(s * SUB_ROWS, SUB_ROWS)])

    return k(rows_pad)


def _sc_propagate(h, rows_pad, cols_pad, zeros_d):
    @functools.partial(
        pl.kernel,
        out_type=jax.ShapeDtypeStruct((NC, N_PAD, D), jnp.float32),
        mesh=_mesh(),
        scratch_types=[
            pltpu.VMEM_SHARED((N_PAD, D), jnp.float32),
            pltpu.VMEM((CH,), jnp.int32),
            pltpu.VMEM((CH,), jnp.int32),
            pltpu.VMEM((CH, D), jnp.float32),
            pltpu.SemaphoreType.DMA,
        ],
    )
    def k(h_hbm, rows_hbm, cols_hbm, zeros_hbm, out_hbm, acc, colv, rowv, rowsb, sem):
        c = lax.axis_index("c")
        s = lax.axis_index("s")
        w = s * NC + c
        pltpu.sync_copy(zeros_hbm, acc.at[pl.ds(s * SUB_ROWS, SUB_ROWS)])
        plsc.subcore_barrier()

        base0 = w * (CPW * CH)

        @pl.loop(0, CPW)
        def _(i):
            base = base0 + i * CH
            pltpu.sync_copy(cols_hbm.at[pl.ds(base, CH)], colv)
            pltpu.sync_copy(rows_hbm.at[pl.ds(base, CH)], rowv)
            pltpu.async_copy(h_hbm.at[colv], rowsb, sem).wait()
            pltpu.sync_copy(rowsb, acc.at[rowv], add=True)

        plsc.subcore_barrier()
        pltpu.sync_copy(acc.at[pl.ds(s * SUB_ROWS, SUB_ROWS)],
                        out_hbm.at[c, pl.ds(s * SUB_ROWS, SUB_ROWS)])

    return k(h, rows_pad, cols_pad, zeros_d)


BN = 1000


def _tc_scale(t_parts, h, deg2):
    def body(t0, t1, h_ref, d0, d1, o_ref):
        d = (d0[0, 0, 0, :] + d1[0, 0, 0, :] + 1.0).reshape(BN, 1)
        o_ref[...] = (t0[0] + t1[0] + h_ref[...]) / d

    return pl.pallas_call(
        body,
        grid=(N // BN,),
        in_specs=[
            pl.BlockSpec((1, BN, D), lambda i: (0, i, 0)),
            pl.BlockSpec((1, BN, D), lambda i: (1, i, 0)),
            pl.BlockSpec((BN, D), lambda i: (i, 0)),
            pl.BlockSpec((1, 1, 1, BN), lambda i: (0, i, 0, 0)),
            pl.BlockSpec((1, 1, 1, BN), lambda i: (1, i, 0, 0)),
        ],
        out_specs=pl.BlockSpec((BN, D), lambda i: (i, 0)),
        out_shape=jax.ShapeDtypeStruct((N, D), jnp.float32),
    )(t_parts, t_parts, h, deg2, deg2)


def _tc_mlp_pool(t_parts, h, deg2, batch3, W1, b1, W2, b2, Wc, bc):
    nblk = N // BN

    def body(t0, t1, h_ref, d0, d1, b_ref, W1r, b1r, W2r, b2r, Wcr, bcr,
             o_ref, accr, cntr):
        i = pl.program_id(0)

        @pl.when(i == 0)
        def _():
            accr[...] = jnp.zeros_like(accr)
            cntr[...] = jnp.zeros_like(cntr)

        d = (d0[0, 0, 0, :] + d1[0, 0, 0, :] + 1.0).reshape(BN, 1)
        h3 = (t0[0] + t1[0] + h_ref[...]) / d
        a = jnp.dot(h3, W1r[...], preferred_element_type=jnp.float32) + b1r[...]
        a = jnp.maximum(a, 0.0)
        bvals = b_ref[...].reshape(1, BN)
        onehot_t = (lax.broadcasted_iota(jnp.int32, (G, BN), 0) == bvals
                    ).astype(jnp.float32)
        accr[...] += jnp.dot(onehot_t, a, preferred_element_type=jnp.float32)
        cntr[...] += jnp.sum(onehot_t, axis=1, keepdims=True)

        @pl.when(i == nblk - 1)
        def _():
            pooled = accr[...] / jnp.maximum(cntr[...], 1.0)
            p2 = jnp.dot(pooled, W2r[...], preferred_element_type=jnp.float32) + b2r[...]
            o_ref[...] = jnp.dot(p2, Wcr[...], preferred_element_type=jnp.float32) + bcr[...]

    return pl.pallas_call(
        body,
        grid=(nblk,),
        in_specs=[
            pl.BlockSpec((1, BN, D), lambda i: (0, i, 0)),
            pl.BlockSpec((1, BN, D), lambda i: (1, i, 0)),
            pl.BlockSpec((BN, D), lambda i: (i, 0)),
            pl.BlockSpec((1, 1, 1, BN), lambda i: (0, i, 0, 0)),
            pl.BlockSpec((1, 1, 1, BN), lambda i: (1, i, 0, 0)),
            pl.BlockSpec((1, 1, BN), lambda i: (i, 0, 0)),
            pl.BlockSpec((D, H), lambda i: (0, 0)),
            pl.BlockSpec((1, H), lambda i: (0, 0)),
            pl.BlockSpec((H, H), lambda i: (0, 0)),
            pl.BlockSpec((1, H), lambda i: (0, 0)),
            pl.BlockSpec((H, C), lambda i: (0, 0)),
            pl.BlockSpec((1, C), lambda i: (0, 0)),
        ],
        out_specs=pl.BlockSpec((G, C), lambda i: (0, 0)),
        out_shape=jax.ShapeDtypeStruct((G, C), jnp.float32),
        scratch_shapes=[
            pltpu.VMEM((G, H), jnp.float32),
            pltpu.VMEM((G, 1), jnp.float32),
        ],
    )(t_parts, t_parts, h, deg2, deg2, batch3,
      W1, b1.reshape(1, H), W2, b2.reshape(1, H), Wc, bc.reshape(1, C))


def kernel(x, edge_index, batch, W1, b1, W2, b2, Wc, bc):
    rows = edge_index[0]
    cols = edge_index[1]
    pad = E_PAD - E
    rows_pad = jnp.concatenate([rows, jnp.full((pad,), DUMMY_ROW, jnp.int32)])
    cols_pad = jnp.concatenate([cols, jnp.zeros((pad,), jnp.int32)])
    zeros_d = jnp.zeros((SUB_ROWS, D), jnp.float32)
    batch3 = batch.reshape(N // BN, 1, BN)

    deg_parts = _sc_degree(rows_pad)
    deg2 = deg_parts[:, :N].reshape(NC, N // BN, 1, BN)
    h = x
    t_parts = None
    for step in range(K):
        t_parts = _sc_propagate(h, rows_pad, cols_pad, zeros_d)
        if step < K - 1:
            h = _tc_scale(t_parts, h, deg2)
    return _tc_mlp_pool(t_parts, h, deg2, batch3, W1, b1, W2, b2, Wc, bc)
